# baseline (speedup 1.0000x reference)
import functools

import jax
import jax.numpy as jnp
from jax import lax
from jax.experimental import pallas as pl
from jax.experimental.pallas import tpu as pltpu
from jax.experimental.pallas import tpu_sc as plsc

_N = 10000
_NP = 10240
_E = 160000
_D = 128
_CHUNK = 128
_NW = 32
_KW = 40
_NCHP = _NW * _KW
_EP = _NCHP * _CHUNK


def _bf(a):
    return a.astype(jnp.bfloat16)


def _dot(a, b):
    return jnp.dot(_bf(a), _bf(b), preferred_element_type=jnp.float32)


def _node_mlp(x, w1, b1, w2, b2, w3, b3):
    bn = 2000

    def body(x_ref, w1r, b1r, w2r, b2r, w3r, b3r, o_ref):
        a = jnp.maximum(_dot(x_ref[...], w1r[...]) + b1r[...], 0.0)
        b = jnp.maximum(_dot(a, w2r[...]) + b2r[...], 0.0)
        o_ref[...] = _dot(b, w3r[...]) + b3r[...]

    ws = (w1, b1, w2, b2, w3, b3)
    return pl.pallas_call(
        body,
        grid=(_N // bn,),
        in_specs=[pl.BlockSpec((bn, _D), lambda i: (i, 0))]
        + [pl.BlockSpec(w.shape, lambda i: (0,) * w.ndim) for w in ws],
        out_specs=pl.BlockSpec((bn, _D), lambda i: (i, 0)),
        out_shape=jax.ShapeDtypeStruct((_NP, _D), jnp.float32),
    )(x, *ws)


def _sc_gather(hb, src, dst):
    mesh = plsc.VectorSubcoreMesh(core_axis_name="c", subcore_axis_name="s")
    out_t = tuple(
        jax.ShapeDtypeStruct((_NCHP, _CHUNK, _D), jnp.float32) for _ in range(2)
    )
    nbuf = 2
    nq = _KW // nbuf
    stage = _NP // 16

    @functools.partial(
        pl.kernel,
        out_type=out_t,
        mesh=mesh,
        scratch_types=[
            pltpu.VMEM_SHARED((_NP, _D), jnp.float32),
            pltpu.VMEM((_KW, _CHUNK), jnp.int32),
            [pltpu.VMEM((_CHUNK, _D), jnp.float32) for _ in range(nbuf)],
            [pltpu.SemaphoreType.DMA for _ in range(nbuf)],
            [pltpu.SemaphoreType.DMA for _ in range(nbuf)],
        ],
    )
    def gk(h_hbm, i1, i2, o1, o2, hsp, idx_v, rows, sem_g, sem_o):
        w = lax.axis_index("s") * 2 + lax.axis_index("c")
        s = lax.axis_index("s")
        pltpu.sync_copy(h_hbm.at[pl.ds(s * stage, stage)],
                        hsp.at[pl.ds(s * stage, stage)])
        plsc.subcore_barrier()
        for idx_hbm, out_hbm in ((i1, o1), (i2, o2)):
            pltpu.sync_copy(idx_hbm.at[w], idx_v)

            def body(q, carry, out_hbm=out_hbm):
                descs = []
                for b in range(nbuf):
                    k = q * nbuf + b

                    @pl.when(q > 0)
                    def _(b=b, k=k):
                        pltpu.make_async_copy(
                            rows[b], out_hbm.at[w * _KW + k - nbuf], sem_o[b]
                        ).wait()

                    descs.append(
                        pltpu.async_copy(hsp.at[idx_v.at[k]], rows[b], sem_g[b])
                    )
                for b in range(nbuf):
                    descs[b].wait()
                for b in range(nbuf):
                    k = q * nbuf + b
                    pltpu.async_copy(rows[b], out_hbm.at[w * _KW + k], sem_o[b])
                return carry

            lax.fori_loop(0, nq, body, 0)
            for b in range(nbuf):
                pltpu.make_async_copy(
                    rows[b], out_hbm.at[w * _KW + _KW - nbuf + b], sem_o[b]
                ).wait()

    return gk(hb, src, dst)


def _sc_counts(d1, d0, zrows, ones):
    mesh = plsc.VectorSubcoreMesh(core_axis_name="c", subcore_axis_name="s")
    rows_per_tile = _NP // 16
    kt = _NCHP // 16
    out_t = jax.ShapeDtypeStruct((2, _NP, _D), jnp.float32)

    @functools.partial(
        pl.kernel,
        out_type=out_t,
        mesh=mesh,
        scratch_types=[
            pltpu.VMEM_SHARED((_NP, _D), jnp.float32),
            pltpu.VMEM((kt, _CHUNK), jnp.int32),
            pltpu.VMEM((_CHUNK, _D), jnp.float32),
            [pltpu.SemaphoreType.DMA for _ in range(2)],
        ],
    )
    def ck(d1h, d0h, zh, oneh, oh, acc, idx_v, val_v, sem_a):
        c = lax.axis_index("c")
        s = lax.axis_index("s")
        my_rows = pl.ds(s * rows_per_tile, rows_per_tile)
        pltpu.sync_copy(oneh, val_v)
        pltpu.sync_copy(zh, acc.at[my_rows])
        for cc, dh in ((0, d1h), (1, d0h)):

            @pl.when(c == cc)
            def _(dh=dh):
                pltpu.sync_copy(dh.at[s], idx_v)

        plsc.subcore_barrier()

        def body(q, carry):
            for b in range(2):
                k = q * 2 + b

                @pl.when(q > 0)
                def _(b=b, k=k):
                    pltpu.make_async_copy(
                        val_v, acc.at[idx_v.at[k]], sem_a[b]
                    ).wait()

                pltpu.async_copy(val_v, acc.at[idx_v.at[k]], sem_a[b], add=True)
            return carry

        lax.fori_loop(0, kt // 2, body, 0)
        for b in range(2):
            pltpu.make_async_copy(
                val_v, acc.at[idx_v.at[kt - 2 + b]], sem_a[b]
            ).wait()
        plsc.subcore_barrier()
        pltpu.sync_copy(acc.at[my_rows], oh.at[c, my_rows])

    return ck(d1, d0, zrows, ones)


def _sc_scatter(u, d, zrows):
    mesh = plsc.VectorSubcoreMesh(core_axis_name="c", subcore_axis_name="s")
    rows_per_tile = _NP // 16
    nbuf = 2
    nq = _KW // nbuf
    out_t = jax.ShapeDtypeStruct((2, _NP, _D), jnp.float32)

    @functools.partial(
        pl.kernel,
        out_type=out_t,
        mesh=mesh,
        scratch_types=[
            pltpu.VMEM_SHARED((_NP, _D), jnp.float32),
            pltpu.VMEM((_KW, _CHUNK), jnp.int32),
            [pltpu.VMEM((_CHUNK, _D), jnp.float32) for _ in range(nbuf)],
            [pltpu.SemaphoreType.DMA for _ in range(nbuf)],
            [pltpu.SemaphoreType.DMA for _ in range(nbuf)],
        ],
    )
    def sk(uh, dh, zh, oh, acc, idx_v, vals, sem_v, sem_a):
        c = lax.axis_index("c")
        s = lax.axis_index("s")
        w = s * 2 + c
        my_rows = pl.ds(s * rows_per_tile, rows_per_tile)
        pltpu.sync_copy(dh.at[w], idx_v)
        pltpu.sync_copy(zh, acc.at[my_rows])
        plsc.subcore_barrier()

        def body(q, carry):
            for b in range(nbuf):
                k = q * nbuf + b

                @pl.when(q > 0)
                def _(b=b, k=k):
                    pltpu.make_async_copy(
                        vals[b], acc.at[idx_v.at[k]], sem_a[b]
                    ).wait()

                pltpu.async_copy(uh.at[w * _KW + k], vals[b], sem_v[b])
            for b in range(nbuf):
                k = q * nbuf + b
                pltpu.make_async_copy(
                    uh.at[w * _KW + k], vals[b], sem_v[b]
                ).wait()
                pltpu.async_copy(
                    vals[b], acc.at[idx_v.at[k]], sem_a[b], add=True
                )
            return carry

        lax.fori_loop(0, nq, body, 0)
        for b in range(nbuf):
            pltpu.make_async_copy(
                vals[b], acc.at[idx_v.at[_KW - nbuf + b]], sem_a[b]
            ).wait()
        plsc.subcore_barrier()
        pltpu.sync_copy(acc.at[my_rows], oh.at[c, my_rows])

    return sk(u, d, zrows)


def _edge_mlp(hs, hd, l, w1e, b1e, w2e, b2e, ew3, eb3, w1a, w1b, w1c, eb1, w2, b2):
    be = 2048

    def body(hs_ref, hd_ref, l_ref, w1er, b1er, w2er, b2er, ew3r, eb3r,
             w1ar, w1br, w1cr, eb1r, w2r, b2r, o_ref):
        z1 = jnp.maximum(l_ref[...] * w1er[...] + b1er[...], 0.0)
        z2 = jnp.maximum(_dot(z1, w2er[...]) + b2er[...], 0.0)
        w1cp = _dot(ew3r[...], w1cr[...])
        c0 = _dot(eb3r[...], w1cr[...]) + eb1r[...]
        l1 = jnp.maximum(
            _dot(hs_ref[...], w1ar[...])
            + _dot(hd_ref[...], w1br[...])
            + _dot(z2, w1cp)
            + c0,
            0.0,
        )
        o_ref[...] = jnp.maximum(_dot(l1, w2r[...]) + b2r[...], 0.0)

    ws = (w1e, b1e, w2e, b2e, ew3, eb3, w1a, w1b, w1c, eb1, w2, b2)
    return pl.pallas_call(
        body,
        grid=(_EP // be,),
        in_specs=[
            pl.BlockSpec((be, _D), lambda i: (i, 0)),
            pl.BlockSpec((be, _D), lambda i: (i, 0)),
            pl.BlockSpec((be, 1), lambda i: (i, 0)),
        ]
        + [pl.BlockSpec(w.shape, lambda i: (0,) * w.ndim) for w in ws],
        out_specs=pl.BlockSpec((be, _D), lambda i: (i, 0)),
        out_shape=jax.ShapeDtypeStruct((_EP, _D), jnp.float32),
    )(hs, hd, l, *ws)


def _avg(s_ref, cnt_ref, ci, w3r, b3r):
    sv = s_ref[0] + s_ref[1]
    cnt = cnt_ref[ci][:, 0:1]
    pos = (cnt > 0.0).astype(jnp.float32)
    return _dot(sv / jnp.maximum(cnt, 1.0), w3r[...]) + b3r[...] * pos


def _aggr_pre(s1, cnts, h, w3_1, b3_1, ga, gb, gb1):
    bn = 2000

    def body(s1_ref, cnt_ref, h_ref, w31r, b31r, gar, gbr, gb1r, o_ref):
        avg1 = _avg(s1_ref, cnt_ref, 0, w31r, b31r)
        o_ref[...] = _dot(h_ref[...], gar[...]) + _dot(avg1, gbr[...]) + gb1r[...]

    ws = (w3_1, b3_1, ga, gb, gb1)
    return pl.pallas_call(
        body,
        grid=(_N // bn,),
        in_specs=[
            pl.BlockSpec((2, bn, _D), lambda i: (0, i, 0)),
            pl.BlockSpec((2, bn, _D), lambda i: (0, i, 0)),
            pl.BlockSpec((bn, _D), lambda i: (i, 0)),
        ]
        + [pl.BlockSpec(w.shape, lambda i: (0,) * w.ndim) for w in ws],
        out_specs=pl.BlockSpec((bn, 256), lambda i: (i, 0)),
        out_shape=jax.ShapeDtypeStruct((_N, 256), jnp.float32),
    )(s1, cnts, h, *ws)


def _aggr(pre, s0, cnts, h, w3_0, b3_0, gc, gw2, gb2, gw3, gb3):
    bn = 2000

    def body(pre_ref, s0_ref, cnt_ref, h_ref, w30r, b30r, gcr, gw2r, gb2r,
             gw3r, gb3r, o_ref):
        hv = h_ref[...]
        avg0 = _avg(s0_ref, cnt_ref, 1, w30r, b30r)
        u1 = jnp.maximum(pre_ref[...] + _dot(avg0, gcr[...]), 0.0)
        u2 = jnp.maximum(_dot(u1, gw2r[...]) + gb2r[...], 0.0)
        o_ref[...] = jnp.maximum(_dot(u2, gw3r[...]) + gb3r[...] + hv, 0.0)

    ws = (w3_0, b3_0, gc, gw2, gb2, gw3, gb3)
    return pl.pallas_call(
        body,
        grid=(_N // bn,),
        in_specs=[
            pl.BlockSpec((bn, 256), lambda i: (i, 0)),
            pl.BlockSpec((2, bn, _D), lambda i: (0, i, 0)),
            pl.BlockSpec((2, bn, _D), lambda i: (0, i, 0)),
            pl.BlockSpec((bn, _D), lambda i: (i, 0)),
        ]
        + [pl.BlockSpec(w.shape, lambda i: (0,) * w.ndim) for w in ws],
        out_specs=pl.BlockSpec((bn, _D), lambda i: (i, 0)),
        out_shape=jax.ShapeDtypeStruct((_N, _D), jnp.float32),
    )(pre, s0, cnts, h, *ws)


def kernel(x, l_e1, l_e0, edge_index_1, edge_index_0, params):
    p = params
    r2 = lambda a: a.reshape(1, -1)

    npad = _EP - _E
    gpad = (jnp.arange(npad, dtype=jnp.int32) * 37) % _N
    spad = _N + jnp.arange(npad, dtype=jnp.int32) % (_NP - _N)
    gidx = lambda a: jnp.concatenate(
        [a.astype(jnp.int32), gpad]).reshape(_NW, _KW, _CHUNK)
    cidx = lambda a: jnp.concatenate(
        [a.astype(jnp.int32), spad]).reshape(16, _NCHP // 16, _CHUNK)
    sidx = lambda a: jnp.concatenate(
        [a.astype(jnp.int32), spad]).reshape(_NW, _KW, _CHUNK)

    src1 = gidx(edge_index_1[0])
    dst1g = gidx(edge_index_1[1])
    src0 = gidx(edge_index_0[0])
    dst0g = gidx(edge_index_0[1])
    dst1c = cidx(edge_index_1[1])
    dst0c = cidx(edge_index_0[1])
    dst1s = sidx(edge_index_1[1])
    dst0s = sidx(edge_index_0[1])
    lpad = jnp.zeros((npad, 1), jnp.float32)
    l1 = jnp.concatenate([l_e1, lpad])
    l0 = jnp.concatenate([l_e0, lpad])

    zrows = jnp.zeros((_NP // 16, _D), jnp.float32)
    ones = jnp.ones((_CHUNK, _D), jnp.float32)

    cnts = _sc_counts(dst1c, dst0c, zrows, ones)

    h = _node_mlp(x, p['v_W1'], r2(p['v_b1']), p['v_W2'], r2(p['v_b2']),
                  p['v_W3'], r2(p['v_b3']))
    ue = (r2(p['e_W1']), r2(p['e_b1']), p['e_W2'], r2(p['e_b2']),
          p['e_W3'], r2(p['e_b3']))

    def emlp(pref, hs, hd, l):
        w1 = p[pref + '_W1']
        return _edge_mlp(
            hs.reshape(_EP, _D), hd.reshape(_EP, _D), l, *ue,
            w1[:_D], w1[_D:2 * _D], w1[2 * _D:], r2(p[pref + '_b1']),
            p[pref + '_W2'], r2(p[pref + '_b2']))

    hs1, hd1 = _sc_gather(h, src1, dst1g)
    hs0, hd0 = _sc_gather(h, src0, dst0g)
    u1 = emlp('edge1', hs1, hd1, l1)
    s1 = _sc_scatter(u1.reshape(_NCHP, _CHUNK, _D), dst1s, zrows)
    u0 = emlp('edge0', hs0, hd0, l0)
    s0 = _sc_scatter(u0.reshape(_NCHP, _CHUNK, _D), dst0s, zrows)

    gw1 = p['aggr_W1']
    pre = _aggr_pre(s1, cnts, h, p['edge1_W3'], r2(p['edge1_b3']),
                    gw1[:_D], gw1[_D:2 * _D], r2(p['aggr_b1']))
    return _aggr(pre, s0, cnts, h,
                 p['edge0_W3'], r2(p['edge0_b3']), gw1[2 * _D:],
                 p['aggr_W2'], r2(p['aggr_b2']), p['aggr_W3'], r2(p['aggr_b3']))


# speedup vs baseline: 1.1184x; 1.1184x over previous
import functools

import jax
import jax.numpy as jnp
from jax import lax
from jax.experimental import pallas as pl
from jax.experimental.pallas import tpu as pltpu
from jax.experimental.pallas import tpu_sc as plsc

_N = 10000
_NP = 10240
_E = 160000
_D = 128
_CHUNK = 128
_NW = 32
_KW = 40
_NCHP = _NW * _KW
_EP = _NCHP * _CHUNK


def _bf(a):
    return a.astype(jnp.bfloat16)


def _dot(a, b):
    return jnp.dot(_bf(a), _bf(b), preferred_element_type=jnp.float32)


def _node_mlp(x, w1, b1, w2, b2, w3, b3):
    bn = 2000

    def body(x_ref, w1r, b1r, w2r, b2r, w3r, b3r, o_ref):
        a = jnp.maximum(_dot(x_ref[...], w1r[...]) + b1r[...], 0.0)
        b = jnp.maximum(_dot(a, w2r[...]) + b2r[...], 0.0)
        o_ref[...] = _dot(b, w3r[...]) + b3r[...]

    ws = (w1, b1, w2, b2, w3, b3)
    return pl.pallas_call(
        body,
        grid=(_N // bn,),
        in_specs=[pl.BlockSpec((bn, _D), lambda i: (i, 0))]
        + [pl.BlockSpec(w.shape, lambda i: (0,) * w.ndim) for w in ws],
        out_specs=pl.BlockSpec((bn, _D), lambda i: (i, 0)),
        out_shape=jax.ShapeDtypeStruct((_NP, _D), jnp.float32),
    )(x, *ws)


def _sc_gather(hb, src, dst):
    mesh = plsc.VectorSubcoreMesh(core_axis_name="c", subcore_axis_name="s")
    out_t = tuple(
        jax.ShapeDtypeStruct((_NCHP, _CHUNK, _D), jnp.float32) for _ in range(2)
    )
    nbuf = 2
    nq = _KW // nbuf
    stage = _NP // 16

    @functools.partial(
        pl.kernel,
        out_type=out_t,
        mesh=mesh,
        scratch_types=[
            pltpu.VMEM_SHARED((_NP, _D), jnp.float32),
            pltpu.VMEM((_KW, _CHUNK), jnp.int32),
            [pltpu.VMEM((_CHUNK, _D), jnp.float32) for _ in range(nbuf)],
            [pltpu.SemaphoreType.DMA for _ in range(nbuf)],
            [pltpu.SemaphoreType.DMA for _ in range(nbuf)],
        ],
    )
    def gk(h_hbm, i1, i2, o1, o2, hsp, idx_v, rows, sem_g, sem_o):
        w = lax.axis_index("s") * 2 + lax.axis_index("c")
        s = lax.axis_index("s")
        pltpu.sync_copy(h_hbm.at[pl.ds(s * stage, stage)],
                        hsp.at[pl.ds(s * stage, stage)])
        plsc.subcore_barrier()
        for idx_hbm, out_hbm in ((i1, o1), (i2, o2)):
            pltpu.sync_copy(idx_hbm.at[w], idx_v)

            def body(q, carry, out_hbm=out_hbm):
                descs = []
                for b in range(nbuf):
                    k = q * nbuf + b

                    @pl.when(q > 0)
                    def _(b=b, k=k):
                        pltpu.make_async_copy(
                            rows[b], out_hbm.at[w * _KW + k - nbuf], sem_o[b]
                        ).wait()

                    descs.append(
                        pltpu.async_copy(hsp.at[idx_v.at[k]], rows[b], sem_g[b])
                    )
                for b in range(nbuf):
                    descs[b].wait()
                for b in range(nbuf):
                    k = q * nbuf + b
                    pltpu.async_copy(rows[b], out_hbm.at[w * _KW + k], sem_o[b])
                return carry

            lax.fori_loop(0, nq, body, 0)
            for b in range(nbuf):
                pltpu.make_async_copy(
                    rows[b], out_hbm.at[w * _KW + _KW - nbuf + b], sem_o[b]
                ).wait()

    return gk(hb, src, dst)


def _sc_counts(d1, d0, zrows, ones):
    mesh = plsc.VectorSubcoreMesh(core_axis_name="c", subcore_axis_name="s")
    rows_per_tile = _NP // 16
    kt = _NCHP // 16
    out_t = jax.ShapeDtypeStruct((2, _NP, _D), jnp.float32)

    @functools.partial(
        pl.kernel,
        out_type=out_t,
        mesh=mesh,
        scratch_types=[
            pltpu.VMEM_SHARED((_NP, _D), jnp.float32),
            pltpu.VMEM((kt, _CHUNK), jnp.int32),
            pltpu.VMEM((_CHUNK, _D), jnp.float32),
            [pltpu.SemaphoreType.DMA for _ in range(2)],
        ],
    )
    def ck(d1h, d0h, zh, oneh, oh, acc, idx_v, val_v, sem_a):
        c = lax.axis_index("c")
        s = lax.axis_index("s")
        my_rows = pl.ds(s * rows_per_tile, rows_per_tile)
        pltpu.sync_copy(oneh, val_v)
        pltpu.sync_copy(zh, acc.at[my_rows])
        for cc, dh in ((0, d1h), (1, d0h)):

            @pl.when(c == cc)
            def _(dh=dh):
                pltpu.sync_copy(dh.at[s], idx_v)

        plsc.subcore_barrier()

        def body(q, carry):
            for b in range(2):
                k = q * 2 + b

                @pl.when(q > 0)
                def _(b=b, k=k):
                    pltpu.make_async_copy(
                        val_v, acc.at[idx_v.at[k]], sem_a[b]
                    ).wait()

                pltpu.async_copy(val_v, acc.at[idx_v.at[k]], sem_a[b], add=True)
            return carry

        lax.fori_loop(0, kt // 2, body, 0)
        for b in range(2):
            pltpu.make_async_copy(
                val_v, acc.at[idx_v.at[kt - 2 + b]], sem_a[b]
            ).wait()
        plsc.subcore_barrier()
        pltpu.sync_copy(acc.at[my_rows], oh.at[c, my_rows])

    return ck(d1, d0, zrows, ones)


def _sc_scatter(u, d, zrows):
    mesh = plsc.VectorSubcoreMesh(core_axis_name="c", subcore_axis_name="s")
    rows_per_tile = _NP // 16
    nbuf = 2
    nq = _KW // nbuf
    out_t = jax.ShapeDtypeStruct((2, _NP, _D), jnp.float32)

    @functools.partial(
        pl.kernel,
        out_type=out_t,
        mesh=mesh,
        scratch_types=[
            pltpu.VMEM_SHARED((_NP, _D), jnp.float32),
            pltpu.VMEM((_KW, _CHUNK), jnp.int32),
            [pltpu.VMEM((_CHUNK, _D), jnp.float32) for _ in range(nbuf)],
            [pltpu.SemaphoreType.DMA for _ in range(nbuf)],
            [pltpu.SemaphoreType.DMA for _ in range(nbuf)],
        ],
    )
    def sk(uh, dh, zh, oh, acc, idx_v, vals, sem_v, sem_a):
        c = lax.axis_index("c")
        s = lax.axis_index("s")
        w = s * 2 + c
        my_rows = pl.ds(s * rows_per_tile, rows_per_tile)
        pltpu.sync_copy(dh.at[w], idx_v)
        pltpu.sync_copy(zh, acc.at[my_rows])
        plsc.subcore_barrier()

        def body(q, carry):
            for b in range(nbuf):
                k = q * nbuf + b

                @pl.when(q > 0)
                def _(b=b, k=k):
                    pltpu.make_async_copy(
                        vals[b], acc.at[idx_v.at[k]], sem_a[b]
                    ).wait()

                pltpu.async_copy(uh.at[w * _KW + k], vals[b], sem_v[b])
            for b in range(nbuf):
                k = q * nbuf + b
                pltpu.make_async_copy(
                    uh.at[w * _KW + k], vals[b], sem_v[b]
                ).wait()
                pltpu.async_copy(
                    vals[b], acc.at[idx_v.at[k]], sem_a[b], add=True
                )
            return carry

        lax.fori_loop(0, nq, body, 0)
        for b in range(nbuf):
            pltpu.make_async_copy(
                vals[b], acc.at[idx_v.at[_KW - nbuf + b]], sem_a[b]
            ).wait()
        plsc.subcore_barrier()
        pltpu.sync_copy(acc.at[my_rows], oh.at[c, my_rows])

    return sk(u, d, zrows)


def _edge_mlp(hs, hd, l, w1e, b1e, w2e, b2e, ew3, eb3, w1a, w1b, w1c, eb1, w2, b2):
    be = 8192

    def body(hs_ref, hd_ref, l_ref, w1er, b1er, w2er, b2er, ew3r, eb3r,
             w1ar, w1br, w1cr, eb1r, w2r, b2r, o_ref):
        z1 = jnp.maximum(l_ref[...] * w1er[...] + b1er[...], 0.0)
        z2 = jnp.maximum(_dot(z1, w2er[...]) + b2er[...], 0.0)
        w1cp = _dot(ew3r[...], w1cr[...])
        c0 = _dot(eb3r[...], w1cr[...]) + eb1r[...]
        l1 = jnp.maximum(
            _dot(hs_ref[...], w1ar[...])
            + _dot(hd_ref[...], w1br[...])
            + _dot(z2, w1cp)
            + c0,
            0.0,
        )
        o_ref[...] = jnp.maximum(_dot(l1, w2r[...]) + b2r[...], 0.0)

    ws = (w1e, b1e, w2e, b2e, ew3, eb3, w1a, w1b, w1c, eb1, w2, b2)
    return pl.pallas_call(
        body,
        grid=(_EP // be,),
        in_specs=[
            pl.BlockSpec((be, _D), lambda i: (i, 0)),
            pl.BlockSpec((be, _D), lambda i: (i, 0)),
            pl.BlockSpec((be, 1), lambda i: (i, 0)),
        ]
        + [pl.BlockSpec(w.shape, lambda i: (0,) * w.ndim) for w in ws],
        out_specs=pl.BlockSpec((be, _D), lambda i: (i, 0)),
        out_shape=jax.ShapeDtypeStruct((_EP, _D), jnp.float32),
    )(hs, hd, l, *ws)


def _avg(s_ref, cnt_ref, ci, w3r, b3r):
    sv = s_ref[0] + s_ref[1]
    cnt = cnt_ref[ci][:, 0:1]
    pos = (cnt > 0.0).astype(jnp.float32)
    return _dot(sv / jnp.maximum(cnt, 1.0), w3r[...]) + b3r[...] * pos


def _aggr_pre(s1, cnts, h, w3_1, b3_1, ga, gb, gb1):
    bn = 2000

    def body(s1_ref, cnt_ref, h_ref, w31r, b31r, gar, gbr, gb1r, o_ref):
        avg1 = _avg(s1_ref, cnt_ref, 0, w31r, b31r)
        o_ref[...] = _dot(h_ref[...], gar[...]) + _dot(avg1, gbr[...]) + gb1r[...]

    ws = (w3_1, b3_1, ga, gb, gb1)
    return pl.pallas_call(
        body,
        grid=(_N // bn,),
        in_specs=[
            pl.BlockSpec((2, bn, _D), lambda i: (0, i, 0)),
            pl.BlockSpec((2, bn, _D), lambda i: (0, i, 0)),
            pl.BlockSpec((bn, _D), lambda i: (i, 0)),
        ]
        + [pl.BlockSpec(w.shape, lambda i: (0,) * w.ndim) for w in ws],
        out_specs=pl.BlockSpec((bn, 256), lambda i: (i, 0)),
        out_shape=jax.ShapeDtypeStruct((_N, 256), jnp.float32),
    )(s1, cnts, h, *ws)


def _aggr(pre, s0, cnts, h, w3_0, b3_0, gc, gw2, gb2, gw3, gb3):
    bn = 2000

    def body(pre_ref, s0_ref, cnt_ref, h_ref, w30r, b30r, gcr, gw2r, gb2r,
             gw3r, gb3r, o_ref):
        hv = h_ref[...]
        avg0 = _avg(s0_ref, cnt_ref, 1, w30r, b30r)
        u1 = jnp.maximum(pre_ref[...] + _dot(avg0, gcr[...]), 0.0)
        u2 = jnp.maximum(_dot(u1, gw2r[...]) + gb2r[...], 0.0)
        o_ref[...] = jnp.maximum(_dot(u2, gw3r[...]) + gb3r[...] + hv, 0.0)

    ws = (w3_0, b3_0, gc, gw2, gb2, gw3, gb3)
    return pl.pallas_call(
        body,
        grid=(_N // bn,),
        in_specs=[
            pl.BlockSpec((bn, 256), lambda i: (i, 0)),
            pl.BlockSpec((2, bn, _D), lambda i: (0, i, 0)),
            pl.BlockSpec((2, bn, _D), lambda i: (0, i, 0)),
            pl.BlockSpec((bn, _D), lambda i: (i, 0)),
        ]
        + [pl.BlockSpec(w.shape, lambda i: (0,) * w.ndim) for w in ws],
        out_specs=pl.BlockSpec((bn, _D), lambda i: (i, 0)),
        out_shape=jax.ShapeDtypeStruct((_N, _D), jnp.float32),
    )(pre, s0, cnts, h, *ws)


def kernel(x, l_e1, l_e0, edge_index_1, edge_index_0, params):
    p = params
    r2 = lambda a: a.reshape(1, -1)

    npad = _EP - _E
    gpad = (jnp.arange(npad, dtype=jnp.int32) * 37) % _N
    spad = _N + jnp.arange(npad, dtype=jnp.int32) % (_NP - _N)
    gidx = lambda a: jnp.concatenate(
        [a.astype(jnp.int32), gpad]).reshape(_NW, _KW, _CHUNK)
    cidx = lambda a: jnp.concatenate(
        [a.astype(jnp.int32), spad]).reshape(16, _NCHP // 16, _CHUNK)
    sidx = lambda a: jnp.concatenate(
        [a.astype(jnp.int32), spad]).reshape(_NW, _KW, _CHUNK)

    src1 = gidx(edge_index_1[0])
    dst1g = gidx(edge_index_1[1])
    src0 = gidx(edge_index_0[0])
    dst0g = gidx(edge_index_0[1])
    dst1c = cidx(edge_index_1[1])
    dst0c = cidx(edge_index_0[1])
    dst1s = sidx(edge_index_1[1])
    dst0s = sidx(edge_index_0[1])
    lpad = jnp.zeros((npad, 1), jnp.float32)
    l1 = jnp.concatenate([l_e1, lpad])
    l0 = jnp.concatenate([l_e0, lpad])

    zrows = jnp.zeros((_NP // 16, _D), jnp.float32)
    ones = jnp.ones((_CHUNK, _D), jnp.float32)

    cnts = _sc_counts(dst1c, dst0c, zrows, ones)

    h = _node_mlp(x, p['v_W1'], r2(p['v_b1']), p['v_W2'], r2(p['v_b2']),
                  p['v_W3'], r2(p['v_b3']))
    ue = (r2(p['e_W1']), r2(p['e_b1']), p['e_W2'], r2(p['e_b2']),
          p['e_W3'], r2(p['e_b3']))

    def emlp(pref, hs, hd, l):
        w1 = p[pref + '_W1']
        return _edge_mlp(
            hs.reshape(_EP, _D), hd.reshape(_EP, _D), l, *ue,
            w1[:_D], w1[_D:2 * _D], w1[2 * _D:], r2(p[pref + '_b1']),
            p[pref + '_W2'], r2(p[pref + '_b2']))

    hs1, hd1 = _sc_gather(h, src1, dst1g)
    hs0, hd0 = _sc_gather(h, src0, dst0g)
    u1 = emlp('edge1', hs1, hd1, l1)
    s1 = _sc_scatter(u1.reshape(_NCHP, _CHUNK, _D), dst1s, zrows)
    u0 = emlp('edge0', hs0, hd0, l0)
    s0 = _sc_scatter(u0.reshape(_NCHP, _CHUNK, _D), dst0s, zrows)

    gw1 = p['aggr_W1']
    pre = _aggr_pre(s1, cnts, h, p['edge1_W3'], r2(p['edge1_b3']),
                    gw1[:_D], gw1[_D:2 * _D], r2(p['aggr_b1']))
    return _aggr(pre, s0, cnts, h,
                 p['edge0_W3'], r2(p['edge0_b3']), gw1[2 * _D:],
                 p['aggr_W2'], r2(p['aggr_b2']), p['aggr_W3'], r2(p['aggr_b3']))
